# Initial kernel scaffold; baseline (speedup 1.0000x reference)
import jax
import jax.numpy as jnp
from jax.experimental import pallas as pl


def kernel(h, coords, a, edge_index, W_e0, b_e0, W_e1, b_e1, W_att, b_att, W_n0, b_n0, W_n1, b_n1, W_c0, b_c0, W_c1, b_c1, W_c2):
    raise NotImplementedError("write your pallas kernel here")


# speedup vs baseline: 3.4036x; 3.4036x over previous
import functools

import jax
import jax.numpy as jnp
from jax import lax
from jax.experimental import pallas as pl
from jax.experimental.pallas import tpu as pltpu
from jax.experimental.pallas import tpu_sc as plsc

NC = 2
NS = 16
NW = NC * NS
CH = 80


def _make_gather(N, E, H):
    per_w = E // NW
    n_ch = per_w // CH
    mesh = plsc.VectorSubcoreMesh(core_axis_name="c", subcore_axis_name="s")

    @functools.partial(
        pl.kernel,
        out_type=(
            jax.ShapeDtypeStruct((E, H), jnp.float32),
            jax.ShapeDtypeStruct((E, H), jnp.float32),
            jax.ShapeDtypeStruct((E, H), jnp.float32),
            jax.ShapeDtypeStruct((E, H), jnp.float32),
        ),
        mesh=mesh,
        scratch_types=[
            pltpu.VMEM((CH,), jnp.int32),
            pltpu.VMEM((CH,), jnp.int32),
            pltpu.VMEM((CH, H), jnp.float32),
            pltpu.VMEM((CH, H), jnp.float32),
            pltpu.VMEM((CH, H), jnp.float32),
            pltpu.VMEM((CH, H), jnp.float32),
            pltpu.SemaphoreType.DMA,
        ],
    )
    def gather_k(h_hbm, c128_hbm, src_hbm, dst_hbm,
                 hs_out, hd_out, cs_out, cd_out,
                 sidx, didx, hs_b, hd_b, cs_b, cd_b, sem):
        wid = lax.axis_index("s") * NC + lax.axis_index("c")
        base0 = wid * per_w

        def body(j, carry):
            base = base0 + j * CH
            pltpu.sync_copy(src_hbm.at[pl.ds(base, CH)], sidx)
            pltpu.sync_copy(dst_hbm.at[pl.ds(base, CH)], didx)
            c1 = pltpu.async_copy(h_hbm.at[sidx], hs_b, sem)
            c2 = pltpu.async_copy(h_hbm.at[didx], hd_b, sem)
            c3 = pltpu.async_copy(c128_hbm.at[sidx], cs_b, sem)
            c4 = pltpu.async_copy(c128_hbm.at[didx], cd_b, sem)
            c1.wait(); c2.wait(); c3.wait(); c4.wait()
            pltpu.sync_copy(hs_b, hs_out.at[pl.ds(base, CH)])
            pltpu.sync_copy(hd_b, hd_out.at[pl.ds(base, CH)])
            pltpu.sync_copy(cs_b, cs_out.at[pl.ds(base, CH)])
            pltpu.sync_copy(cd_b, cd_out.at[pl.ds(base, CH)])
            return carry

        lax.fori_loop(0, n_ch, body, 0)

    return gather_k


def _make_scatter(N, E, H):
    per_t = E // NS
    n_ch = per_t // CH
    rpt = (N // NS) // 8 * 8
    rem = N - NS * rpt
    mesh = plsc.VectorSubcoreMesh(core_axis_name="c", subcore_axis_name="s")

    @functools.partial(
        pl.kernel,
        out_type=(
            jax.ShapeDtypeStruct((N, H), jnp.float32),
            jax.ShapeDtypeStruct((N, H), jnp.float32),
        ),
        mesh=mesh,
        scratch_types=[
            pltpu.VMEM((CH,), jnp.int32),
            pltpu.VMEM((CH, H), jnp.float32),
            pltpu.VMEM_SHARED((N, H), jnp.float32),
        ],
    )
    def scatter_k(msgh_hbm, msgx_hbm, dst_hbm, zh_hbm,
                  hagg_out, xagg_out,
                  didx, m_b, acc):
        cid = lax.axis_index("c")
        sid = lax.axis_index("s")
        base0 = sid * per_t
        r0 = sid * rpt
        pltpu.sync_copy(zh_hbm.at[pl.ds(r0, rpt)], acc.at[pl.ds(r0, rpt)])
        if rem:
            @pl.when(sid == NS - 1)
            def _():
                pltpu.sync_copy(zh_hbm.at[pl.ds(NS * rpt, rem)],
                                acc.at[pl.ds(NS * rpt, rem)])
        plsc.subcore_barrier()

        def make_body(src_ref):
            def body(j, carry):
                base = base0 + j * CH
                pltpu.sync_copy(dst_hbm.at[pl.ds(base, CH)], didx)
                pltpu.sync_copy(src_ref.at[pl.ds(base, CH)], m_b)
                pltpu.sync_copy(m_b, acc.at[didx], add=True)
                return carry
            return body

        @pl.when(cid == 0)
        def _():
            lax.fori_loop(0, n_ch, make_body(msgh_hbm), 0)

        @pl.when(cid == 1)
        def _():
            lax.fori_loop(0, n_ch, make_body(msgx_hbm), 0)

        plsc.subcore_barrier()

        @pl.when(cid == 0)
        def _():
            pltpu.sync_copy(acc.at[pl.ds(r0, rpt)],
                            hagg_out.at[pl.ds(r0, rpt)])
            if rem:
                @pl.when(sid == NS - 1)
                def _():
                    pltpu.sync_copy(acc.at[pl.ds(NS * rpt, rem)],
                                    hagg_out.at[pl.ds(NS * rpt, rem)])

        @pl.when(cid == 1)
        def _():
            pltpu.sync_copy(acc.at[pl.ds(r0, rpt)],
                            xagg_out.at[pl.ds(r0, rpt)])
            if rem:
                @pl.when(sid == NS - 1)
                def _():
                    pltpu.sync_copy(acc.at[pl.ds(NS * rpt, rem)],
                                    xagg_out.at[pl.ds(NS * rpt, rem)])

    return scatter_k


def _edge_block_kernel(hs, hd, cs, cd, a_ref,
                       w1s, w1d, w1a, w1r, b1,
                       we1t, be1, watt, batt,
                       wc1t, bc1, wc2,
                       msgh_out, msgx_out):
    H = hs.shape[1]
    hs_ = hs[...]
    hd_ = hd[...]
    d = cs[...] - cd[...]
    r2 = jnp.sum(d * d, axis=1, keepdims=True)
    r = jnp.sqrt(r2)
    pre = (jnp.dot(hs_, w1s[...], preferred_element_type=jnp.float32)
           + jnp.dot(hd_, w1d[...], preferred_element_type=jnp.float32)
           + jnp.dot(a_ref[...], w1a[...], preferred_element_type=jnp.float32)
           + r * w1r[...] + b1[...])
    pre_e = pre[:, :H]
    pre_c = pre[:, H:]
    m_e = jax.nn.silu(pre_e)
    mh = jax.nn.silu(jnp.dot(m_e, we1t[...],
                             preferred_element_type=jnp.float32) + be1[...])
    att = jax.nn.sigmoid(
        jnp.sum(mh * watt[...], axis=1, keepdims=True) + batt[0, 0])
    msgh_out[...] = att * mh
    m1 = jax.nn.silu(pre_c)
    m2 = jax.nn.silu(jnp.dot(m1, wc1t[...],
                             preferred_element_type=jnp.float32) + bc1[...])
    s = jnp.sum(m2 * wc2[...], axis=1, keepdims=True)
    msgx_out[...] = s * d / (r + 1.0)


def _node_block_kernel(h, c128, hagg, xagg,
                       wn0h, wn0a, bn0, wn1t, bn1,
                       hout, cout):
    h_ = h[...]
    t = jax.nn.silu(jnp.dot(h_, wn0h[...], preferred_element_type=jnp.float32)
                    + jnp.dot(hagg[...], wn0a[...],
                              preferred_element_type=jnp.float32)
                    + bn0[...])
    hout[...] = h_ + jnp.dot(t, wn1t[...],
                             preferred_element_type=jnp.float32) + bn1[...]
    cout[...] = c128[...] + xagg[...]


def kernel(h, coords, a, edge_index, W_e0, b_e0, W_e1, b_e1, W_att, b_att,
           W_n0, b_n0, W_n1, b_n1, W_c0, b_c0, W_c1, b_c1, W_c2):
    N, H = h.shape
    E = a.shape[0]
    DE = a.shape[1]
    f32 = jnp.float32

    src = edge_index[0]
    dst = edge_index[1]
    c128 = jnp.pad(coords, ((0, 0), (0, H - coords.shape[1])))

    hs, hd, cs, cd = _make_gather(N, E, H)(h, c128, src, dst)

    w1s = jnp.concatenate([W_e0[:, :H], W_c0[:, :H]], axis=0).T
    w1d = jnp.concatenate([W_e0[:, H:2 * H], W_c0[:, H:2 * H]], axis=0).T
    w1a = jnp.concatenate([W_e0[:, 2 * H + 1:], W_c0[:, 2 * H + 1:]],
                          axis=0).T
    w1r = jnp.concatenate([W_e0[:, 2 * H], W_c0[:, 2 * H]])[None, :]
    b1 = jnp.concatenate([b_e0, b_c0])[None, :]
    we1t = W_e1.T
    be1 = b_e1[None, :]
    watt = W_att
    batt = b_att[None, :]
    wc1t = W_c1.T
    bc1 = b_c1[None, :]
    wc2 = W_c2

    BE = 2000
    n_eb = E // BE
    full = lambda shape: pl.BlockSpec(shape, lambda i: (0,) * len(shape))
    eb = lambda w: pl.BlockSpec((BE, w), lambda i: (i, 0))
    msgh, msgx = pl.pallas_call(
        _edge_block_kernel,
        grid=(n_eb,),
        in_specs=[
            eb(H), eb(H), eb(H), eb(H), eb(DE),
            full((H, 2 * H)), full((H, 2 * H)), full((DE, 2 * H)),
            full((1, 2 * H)), full((1, 2 * H)),
            full((H, H)), full((1, H)), full((1, H)), full((1, 1)),
            full((H, H)), full((1, H)), full((1, H)),
        ],
        out_specs=[eb(H), eb(H)],
        out_shape=[
            jax.ShapeDtypeStruct((E, H), f32),
            jax.ShapeDtypeStruct((E, H), f32),
        ],
    )(hs, hd, cs, cd, a, w1s, w1d, w1a, w1r, b1,
      we1t, be1, watt, batt, wc1t, bc1, wc2)

    zh = jnp.zeros((N, H), f32)
    hagg, xagg = _make_scatter(N, E, H)(msgh, msgx, dst, zh)

    wn0h = W_n0[:, :H].T
    wn0a = W_n0[:, H:].T
    bn0 = b_n0[None, :]
    wn1t = W_n1.T
    bn1 = b_n1[None, :]
    BN = 2000
    n_nb = N // BN
    nb = lambda w: pl.BlockSpec((BN, w), lambda i: (i, 0))
    hout, cout128 = pl.pallas_call(
        _node_block_kernel,
        grid=(n_nb,),
        in_specs=[
            nb(H), nb(H), nb(H), nb(H),
            full((H, H)), full((H, H)), full((1, H)),
            full((H, H)), full((1, H)),
        ],
        out_specs=[nb(H), nb(H)],
        out_shape=[
            jax.ShapeDtypeStruct((N, H), f32),
            jax.ShapeDtypeStruct((N, H), f32),
        ],
    )(h, c128, hagg, xagg, wn0h, wn0a, bn0, wn1t, bn1)

    return hout, cout128[:, :coords.shape[1]]


# speedup vs baseline: 3.4229x; 1.0057x over previous
import functools

import jax
import jax.numpy as jnp
from jax import lax
from jax.experimental import pallas as pl
from jax.experimental.pallas import tpu as pltpu
from jax.experimental.pallas import tpu_sc as plsc

NC = 2
NS = 16
NW = NC * NS
CH = 80


def _make_gather(N, E, H):
    per_w = E // NW
    n_ch = per_w // CH
    mesh = plsc.VectorSubcoreMesh(core_axis_name="c", subcore_axis_name="s")

    @functools.partial(
        pl.kernel,
        out_type=(
            jax.ShapeDtypeStruct((E, H), jnp.float32),
            jax.ShapeDtypeStruct((E, H), jnp.float32),
            jax.ShapeDtypeStruct((E, H), jnp.float32),
        ),
        mesh=mesh,
        scratch_types=[
            pltpu.VMEM((CH,), jnp.int32),
            pltpu.VMEM((CH,), jnp.int32),
            pltpu.VMEM((CH, H), jnp.float32),
            pltpu.VMEM((CH, H), jnp.float32),
            pltpu.VMEM((CH, H), jnp.float32),
            pltpu.VMEM((CH, H), jnp.float32),
            pltpu.SemaphoreType.DMA,
        ],
    )
    def gather_k(h_hbm, c128_hbm, src_hbm, dst_hbm,
                 hs_out, hd_out, d_out,
                 sidx, didx, hs_b, hd_b, cs_b, cd_b, sem):
        wid = lax.axis_index("s") * NC + lax.axis_index("c")
        base0 = wid * per_w

        def body(j, carry):
            base = base0 + j * CH
            pltpu.sync_copy(src_hbm.at[pl.ds(base, CH)], sidx)
            pltpu.sync_copy(dst_hbm.at[pl.ds(base, CH)], didx)
            c1 = pltpu.async_copy(h_hbm.at[sidx], hs_b, sem)
            c2 = pltpu.async_copy(h_hbm.at[didx], hd_b, sem)
            c3 = pltpu.async_copy(c128_hbm.at[sidx], cs_b, sem)
            c4 = pltpu.async_copy(c128_hbm.at[didx], cd_b, sem)
            c1.wait(); c2.wait(); c3.wait(); c4.wait()

            def sub_row(i, c):
                cs_b[i, pl.ds(0, 16)] = (cs_b[i, pl.ds(0, 16)]
                                         - cd_b[i, pl.ds(0, 16)])
                return c
            lax.fori_loop(0, CH, sub_row, 0)

            pltpu.sync_copy(hs_b, hs_out.at[pl.ds(base, CH)])
            pltpu.sync_copy(hd_b, hd_out.at[pl.ds(base, CH)])
            pltpu.sync_copy(cs_b, d_out.at[pl.ds(base, CH)])
            return carry

        lax.fori_loop(0, n_ch, body, 0)

    return gather_k


def _make_scatter(N, E, H):
    per_t = E // NS
    n_ch = per_t // CH
    rpt = (N // NS) // 8 * 8
    rem = N - NS * rpt
    mesh = plsc.VectorSubcoreMesh(core_axis_name="c", subcore_axis_name="s")

    @functools.partial(
        pl.kernel,
        out_type=(
            jax.ShapeDtypeStruct((N, H), jnp.float32),
            jax.ShapeDtypeStruct((N, H), jnp.float32),
        ),
        mesh=mesh,
        scratch_types=[
            pltpu.VMEM((CH,), jnp.int32),
            pltpu.VMEM((CH, H), jnp.float32),
            pltpu.VMEM_SHARED((N, H), jnp.float32),
        ],
    )
    def scatter_k(msgh_hbm, msgx_hbm, dst_hbm, zh_hbm,
                  hagg_out, xagg_out,
                  didx, m_b, acc):
        cid = lax.axis_index("c")
        sid = lax.axis_index("s")
        base0 = sid * per_t
        r0 = sid * rpt
        pltpu.sync_copy(zh_hbm.at[pl.ds(r0, rpt)], acc.at[pl.ds(r0, rpt)])
        if rem:
            @pl.when(sid == NS - 1)
            def _():
                pltpu.sync_copy(zh_hbm.at[pl.ds(NS * rpt, rem)],
                                acc.at[pl.ds(NS * rpt, rem)])
        plsc.subcore_barrier()

        def make_body(src_ref):
            def body(j, carry):
                base = base0 + j * CH
                pltpu.sync_copy(dst_hbm.at[pl.ds(base, CH)], didx)
                pltpu.sync_copy(src_ref.at[pl.ds(base, CH)], m_b)
                pltpu.sync_copy(m_b, acc.at[didx], add=True)
                return carry
            return body

        @pl.when(cid == 0)
        def _():
            lax.fori_loop(0, n_ch, make_body(msgh_hbm), 0)

        @pl.when(cid == 1)
        def _():
            lax.fori_loop(0, n_ch, make_body(msgx_hbm), 0)

        plsc.subcore_barrier()

        @pl.when(cid == 0)
        def _():
            pltpu.sync_copy(acc.at[pl.ds(r0, rpt)],
                            hagg_out.at[pl.ds(r0, rpt)])
            if rem:
                @pl.when(sid == NS - 1)
                def _():
                    pltpu.sync_copy(acc.at[pl.ds(NS * rpt, rem)],
                                    hagg_out.at[pl.ds(NS * rpt, rem)])

        @pl.when(cid == 1)
        def _():
            pltpu.sync_copy(acc.at[pl.ds(r0, rpt)],
                            xagg_out.at[pl.ds(r0, rpt)])
            if rem:
                @pl.when(sid == NS - 1)
                def _():
                    pltpu.sync_copy(acc.at[pl.ds(NS * rpt, rem)],
                                    xagg_out.at[pl.ds(NS * rpt, rem)])

    return scatter_k


def _edge_block_kernel(hs, hd, d_ref, a_ref,
                       w1s, w1d, w1a, w1r, b1,
                       we1t, be1, watt, batt,
                       wc1t, bc1, wc2,
                       msgh_out, msgx_out):
    H = hs.shape[1]
    bf16 = jnp.bfloat16
    f32 = jnp.float32
    hs_ = hs[...].astype(bf16)
    hd_ = hd[...].astype(bf16)
    d = d_ref[...]
    r2 = jnp.sum(d * d, axis=1, keepdims=True)
    r = jnp.sqrt(r2)
    pre = (jnp.dot(hs_, w1s[...], preferred_element_type=f32)
           + jnp.dot(hd_, w1d[...], preferred_element_type=f32)
           + jnp.dot(a_ref[...], w1a[...], preferred_element_type=f32)
           + r * w1r[...] + b1[...])
    pre_e = pre[:, :H]
    pre_c = pre[:, H:]
    m_e = jax.nn.silu(pre_e).astype(bf16)
    mh = jax.nn.silu(jnp.dot(m_e, we1t[...],
                             preferred_element_type=f32) + be1[...])
    att = jax.nn.sigmoid(
        jnp.sum(mh * watt[...], axis=1, keepdims=True) + batt[0, 0])
    msgh_out[...] = att * mh
    m1 = jax.nn.silu(pre_c).astype(bf16)
    m2 = jax.nn.silu(jnp.dot(m1, wc1t[...],
                             preferred_element_type=f32) + bc1[...])
    s = jnp.sum(m2 * wc2[...], axis=1, keepdims=True)
    msgx_out[...] = s * d / (r + 1.0)


def _node_block_kernel(h, c128, hagg, xagg,
                       wn0h, wn0a, bn0, wn1t, bn1,
                       hout, cout):
    h_ = h[...]
    t = jax.nn.silu(jnp.dot(h_, wn0h[...], preferred_element_type=jnp.float32)
                    + jnp.dot(hagg[...], wn0a[...],
                              preferred_element_type=jnp.float32)
                    + bn0[...])
    hout[...] = h_ + jnp.dot(t, wn1t[...],
                             preferred_element_type=jnp.float32) + bn1[...]
    cout[...] = c128[...] + xagg[...]


def kernel(h, coords, a, edge_index, W_e0, b_e0, W_e1, b_e1, W_att, b_att,
           W_n0, b_n0, W_n1, b_n1, W_c0, b_c0, W_c1, b_c1, W_c2):
    N, H = h.shape
    E = a.shape[0]
    DE = a.shape[1]
    f32 = jnp.float32

    bf16 = jnp.bfloat16
    src = edge_index[0]
    dst = edge_index[1]
    c128 = jnp.pad(coords, ((0, 0), (0, H - coords.shape[1])))

    hs, hd, d = _make_gather(N, E, H)(h, c128, src, dst)

    w1s = jnp.concatenate([W_e0[:, :H], W_c0[:, :H]],
                          axis=0).T.astype(bf16)
    w1d = jnp.concatenate([W_e0[:, H:2 * H], W_c0[:, H:2 * H]],
                          axis=0).T.astype(bf16)
    w1a = jnp.concatenate([W_e0[:, 2 * H + 1:], W_c0[:, 2 * H + 1:]],
                          axis=0).T.astype(bf16)
    w1r = jnp.concatenate([W_e0[:, 2 * H], W_c0[:, 2 * H]])[None, :]
    b1 = jnp.concatenate([b_e0, b_c0])[None, :]
    we1t = W_e1.T.astype(bf16)
    be1 = b_e1[None, :]
    watt = W_att
    batt = b_att[None, :]
    wc1t = W_c1.T.astype(bf16)
    bc1 = b_c1[None, :]
    wc2 = W_c2
    a_bf = a.astype(bf16)

    BE = 2000
    n_eb = E // BE
    full = lambda shape: pl.BlockSpec(shape, lambda i: (0,) * len(shape))
    eb = lambda w: pl.BlockSpec((BE, w), lambda i: (i, 0))
    msgh, msgx = pl.pallas_call(
        _edge_block_kernel,
        grid=(n_eb,),
        in_specs=[
            eb(H), eb(H), eb(H), eb(DE),
            full((H, 2 * H)), full((H, 2 * H)), full((DE, 2 * H)),
            full((1, 2 * H)), full((1, 2 * H)),
            full((H, H)), full((1, H)), full((1, H)), full((1, 1)),
            full((H, H)), full((1, H)), full((1, H)),
        ],
        out_specs=[eb(H), eb(H)],
        out_shape=[
            jax.ShapeDtypeStruct((E, H), f32),
            jax.ShapeDtypeStruct((E, H), f32),
        ],
    )(hs, hd, d, a_bf, w1s, w1d, w1a, w1r, b1,
      we1t, be1, watt, batt, wc1t, bc1, wc2)

    zh = jnp.zeros((N, H), f32)
    hagg, xagg = _make_scatter(N, E, H)(msgh, msgx, dst, zh)

    wn0h = W_n0[:, :H].T
    wn0a = W_n0[:, H:].T
    bn0 = b_n0[None, :]
    wn1t = W_n1.T
    bn1 = b_n1[None, :]
    BN = 2000
    n_nb = N // BN
    nb = lambda w: pl.BlockSpec((BN, w), lambda i: (i, 0))
    hout, cout128 = pl.pallas_call(
        _node_block_kernel,
        grid=(n_nb,),
        in_specs=[
            nb(H), nb(H), nb(H), nb(H),
            full((H, H)), full((H, H)), full((1, H)),
            full((H, H)), full((1, H)),
        ],
        out_specs=[nb(H), nb(H)],
        out_shape=[
            jax.ShapeDtypeStruct((N, H), f32),
            jax.ShapeDtypeStruct((N, H), f32),
        ],
    )(h, c128, hagg, xagg, wn0h, wn0a, bn0, wn1t, bn1)

    return hout, cout128[:, :coords.shape[1]]


# speedup vs baseline: 4.7836x; 1.3975x over previous
import functools

import jax
import jax.numpy as jnp
from jax import lax
from jax.experimental import pallas as pl
from jax.experimental.pallas import tpu as pltpu
from jax.experimental.pallas import tpu_sc as plsc

NC = 2
NS = 16
NW = NC * NS
CH = 80


def _make_gather(N, E, H):
    per_w = E // NW
    n_ch = per_w // CH
    mesh = plsc.VectorSubcoreMesh(core_axis_name="c", subcore_axis_name="s")

    @functools.partial(
        pl.kernel,
        out_type=(
            jax.ShapeDtypeStruct((E, H), jnp.float32),
            jax.ShapeDtypeStruct((E, H), jnp.float32),
            jax.ShapeDtypeStruct((E, H), jnp.float32),
        ),
        mesh=mesh,
        scratch_types=[
            [pltpu.VMEM((CH,), jnp.int32)] * 2,
            [pltpu.VMEM((CH,), jnp.int32)] * 2,
            [pltpu.VMEM((CH, H), jnp.float32)] * 2,
            [pltpu.VMEM((CH, H), jnp.float32)] * 2,
            [pltpu.VMEM((CH, H), jnp.float32)] * 2,
            [pltpu.VMEM((CH, H), jnp.float32)] * 2,
            [pltpu.SemaphoreType.DMA] * 2,
            [pltpu.SemaphoreType.DMA] * 2,
            [pltpu.SemaphoreType.DMA] * 2,
        ],
    )
    def gather_k(h_hbm, c128_hbm, src_hbm, dst_hbm,
                 hs_out, hd_out, d_out,
                 sidx, didx, hs_b, hd_b, cs_b, cd_b,
                 sem_l, sem_g, sem_w):
        wid = lax.axis_index("s") * NC + lax.axis_index("c")
        base0 = wid * per_w

        def fire_l(b, j):
            base = base0 + j * CH
            pltpu.async_copy(src_hbm.at[pl.ds(base, CH)], sidx[b], sem_l[b])
            pltpu.async_copy(dst_hbm.at[pl.ds(base, CH)], didx[b], sem_l[b])

        def wait_l(b):
            pltpu.make_async_copy(src_hbm.at[pl.ds(0, CH)], sidx[b],
                                  sem_l[b]).wait()
            pltpu.make_async_copy(dst_hbm.at[pl.ds(0, CH)], didx[b],
                                  sem_l[b]).wait()

        def fire_g(b):
            pltpu.async_copy(h_hbm.at[sidx[b]], hs_b[b], sem_g[b])
            pltpu.async_copy(h_hbm.at[didx[b]], hd_b[b], sem_g[b])
            pltpu.async_copy(c128_hbm.at[sidx[b]], cs_b[b], sem_g[b])
            pltpu.async_copy(c128_hbm.at[didx[b]], cd_b[b], sem_g[b])

        def wait_g(b):
            pltpu.make_async_copy(h_hbm.at[sidx[b]], hs_b[b], sem_g[b]).wait()
            pltpu.make_async_copy(h_hbm.at[didx[b]], hd_b[b], sem_g[b]).wait()
            pltpu.make_async_copy(c128_hbm.at[sidx[b]], cs_b[b],
                                  sem_g[b]).wait()
            pltpu.make_async_copy(c128_hbm.at[didx[b]], cd_b[b],
                                  sem_g[b]).wait()

        def diffs(b):
            def sub_row(i, c):
                cs_b[b][i, pl.ds(0, 16)] = (cs_b[b][i, pl.ds(0, 16)]
                                            - cd_b[b][i, pl.ds(0, 16)])
                return c
            lax.fori_loop(0, CH, sub_row, 0)

        def fire_w(b, j):
            base = base0 + j * CH
            pltpu.async_copy(hs_b[b], hs_out.at[pl.ds(base, CH)], sem_w[b])
            pltpu.async_copy(hd_b[b], hd_out.at[pl.ds(base, CH)], sem_w[b])
            pltpu.async_copy(cs_b[b], d_out.at[pl.ds(base, CH)], sem_w[b])

        def wait_w(b):
            z = pl.ds(0, CH)
            pltpu.make_async_copy(hs_b[b], hs_out.at[z], sem_w[b]).wait()
            pltpu.make_async_copy(hd_b[b], hd_out.at[z], sem_w[b]).wait()
            pltpu.make_async_copy(cs_b[b], d_out.at[z], sem_w[b]).wait()

        fire_l(0, 0)
        wait_l(0); fire_g(0)
        fire_l(1, 1)
        wait_g(0); fire_l(0, 2); diffs(0); fire_w(0, 0)
        wait_l(1); fire_g(1)
        wait_g(1); fire_l(1, 3); diffs(1); fire_w(1, 1)

        n_pairs = (n_ch - 2) // 2
        last = n_ch - 1

        def body(k, carry):
            for b in (0, 1):
                j = 2 * k + b
                wait_l(b)
                wait_w(b)
                fire_g(b)
                wait_g(b)
                jn = jnp.minimum(j + 2, last)
                fire_l(b, jn)
                diffs(b)
                fire_w(b, j)
            return carry

        lax.fori_loop(1, 1 + n_pairs, body, 0)

        if n_ch % 2:
            wait_l(0)
            wait_w(0)
            fire_g(0)
            wait_g(0)
            diffs(0)
            fire_w(0, last)
            wait_l(1)
            wait_w(1)
            wait_w(0)
        else:
            wait_l(0); wait_l(1)
            wait_w(0); wait_w(1)

    return gather_k


def _make_scatter(N, E, H):
    per_t = E // NS
    n_ch = per_t // CH
    rpt = (N // NS) // 8 * 8
    rem = N - NS * rpt
    mesh = plsc.VectorSubcoreMesh(core_axis_name="c", subcore_axis_name="s")

    @functools.partial(
        pl.kernel,
        out_type=(
            jax.ShapeDtypeStruct((N, H), jnp.float32),
            jax.ShapeDtypeStruct((N, H), jnp.float32),
        ),
        mesh=mesh,
        scratch_types=[
            [pltpu.VMEM((CH,), jnp.int32)] * 2,
            [pltpu.VMEM((CH, H), jnp.float32)] * 2,
            pltpu.VMEM_SHARED((N, H), jnp.float32),
            [pltpu.SemaphoreType.DMA] * 2,
            [pltpu.SemaphoreType.DMA] * 2,
        ],
    )
    def scatter_k(msgh_hbm, msgx_hbm, dst_hbm, zh_hbm,
                  hagg_out, xagg_out,
                  didx, m_b, acc, sem_l, sem_a):
        cid = lax.axis_index("c")
        sid = lax.axis_index("s")
        base0 = sid * per_t
        r0 = sid * rpt
        pltpu.sync_copy(zh_hbm.at[pl.ds(r0, rpt)], acc.at[pl.ds(r0, rpt)])
        if rem:
            @pl.when(sid == NS - 1)
            def _():
                pltpu.sync_copy(zh_hbm.at[pl.ds(NS * rpt, rem)],
                                acc.at[pl.ds(NS * rpt, rem)])
        plsc.subcore_barrier()

        def run_pipeline(src_ref):
            def fire_l(b, j):
                base = base0 + j * CH
                pltpu.async_copy(dst_hbm.at[pl.ds(base, CH)], didx[b],
                                 sem_l[b])
                pltpu.async_copy(src_ref.at[pl.ds(base, CH)], m_b[b],
                                 sem_l[b])

            def wait_l(b):
                pltpu.make_async_copy(dst_hbm.at[pl.ds(0, CH)], didx[b],
                                      sem_l[b]).wait()
                pltpu.make_async_copy(src_ref.at[pl.ds(0, CH)], m_b[b],
                                      sem_l[b]).wait()

            def fire_a(b):
                pltpu.async_copy(m_b[b], acc.at[didx[b]], sem_a[b], add=True)

            def wait_a(b):
                pltpu.make_async_copy(m_b[b], acc.at[didx[b]],
                                      sem_a[b]).wait()

            fire_l(0, 0)
            fire_l(1, 1)
            last = n_ch - 1

            def body(k, carry):
                for b in (0, 1):
                    j = 2 * k + b
                    wait_l(b)
                    fire_a(b)
                    wait_a(b)
                    jn = jnp.minimum(j + 2, last)
                    fire_l(b, jn)
                return carry

            lax.fori_loop(0, n_ch // 2, body, 0)
            wait_l(0)
            wait_l(1)

        @pl.when(cid == 0)
        def _():
            run_pipeline(msgh_hbm)

        @pl.when(cid == 1)
        def _():
            run_pipeline(msgx_hbm)

        plsc.subcore_barrier()

        @pl.when(cid == 0)
        def _():
            pltpu.sync_copy(acc.at[pl.ds(r0, rpt)],
                            hagg_out.at[pl.ds(r0, rpt)])
            if rem:
                @pl.when(sid == NS - 1)
                def _():
                    pltpu.sync_copy(acc.at[pl.ds(NS * rpt, rem)],
                                    hagg_out.at[pl.ds(NS * rpt, rem)])

        @pl.when(cid == 1)
        def _():
            pltpu.sync_copy(acc.at[pl.ds(r0, rpt)],
                            xagg_out.at[pl.ds(r0, rpt)])
            if rem:
                @pl.when(sid == NS - 1)
                def _():
                    pltpu.sync_copy(acc.at[pl.ds(NS * rpt, rem)],
                                    xagg_out.at[pl.ds(NS * rpt, rem)])

    return scatter_k


def _edge_block_kernel(hs, hd, d_ref, a_ref,
                       w1s, w1d, w1a, w1r, b1,
                       we1t, be1, watt, batt,
                       wc1t, bc1, wc2,
                       msgh_out, msgx_out):
    H = hs.shape[1]
    bf16 = jnp.bfloat16
    f32 = jnp.float32
    hs_ = hs[...].astype(bf16)
    hd_ = hd[...].astype(bf16)
    d = d_ref[...]
    r2 = jnp.sum(d * d, axis=1, keepdims=True)
    r = jnp.sqrt(r2)
    pre = (jnp.dot(hs_, w1s[...], preferred_element_type=f32)
           + jnp.dot(hd_, w1d[...], preferred_element_type=f32)
           + jnp.dot(a_ref[...], w1a[...], preferred_element_type=f32)
           + r * w1r[...] + b1[...])
    pre_e = pre[:, :H]
    pre_c = pre[:, H:]
    m_e = jax.nn.silu(pre_e).astype(bf16)
    mh = jax.nn.silu(jnp.dot(m_e, we1t[...],
                             preferred_element_type=f32) + be1[...])
    att = jax.nn.sigmoid(
        jnp.sum(mh * watt[...], axis=1, keepdims=True) + batt[0, 0])
    msgh_out[...] = att * mh
    m1 = jax.nn.silu(pre_c).astype(bf16)
    m2 = jax.nn.silu(jnp.dot(m1, wc1t[...],
                             preferred_element_type=f32) + bc1[...])
    s = jnp.sum(m2 * wc2[...], axis=1, keepdims=True)
    msgx_out[...] = s * d / (r + 1.0)


def _node_block_kernel(h, c128, hagg, xagg,
                       wn0h, wn0a, bn0, wn1t, bn1,
                       hout, cout):
    h_ = h[...]
    t = jax.nn.silu(jnp.dot(h_, wn0h[...], preferred_element_type=jnp.float32)
                    + jnp.dot(hagg[...], wn0a[...],
                              preferred_element_type=jnp.float32)
                    + bn0[...])
    hout[...] = h_ + jnp.dot(t, wn1t[...],
                             preferred_element_type=jnp.float32) + bn1[...]
    cout[...] = c128[...] + xagg[...]


def kernel(h, coords, a, edge_index, W_e0, b_e0, W_e1, b_e1, W_att, b_att,
           W_n0, b_n0, W_n1, b_n1, W_c0, b_c0, W_c1, b_c1, W_c2):
    N, H = h.shape
    E = a.shape[0]
    DE = a.shape[1]
    f32 = jnp.float32

    bf16 = jnp.bfloat16
    src = edge_index[0]
    dst = edge_index[1]
    c128 = jnp.pad(coords, ((0, 0), (0, H - coords.shape[1])))

    hs, hd, d = _make_gather(N, E, H)(h, c128, src, dst)

    w1s = jnp.concatenate([W_e0[:, :H], W_c0[:, :H]],
                          axis=0).T.astype(bf16)
    w1d = jnp.concatenate([W_e0[:, H:2 * H], W_c0[:, H:2 * H]],
                          axis=0).T.astype(bf16)
    w1a = jnp.concatenate([W_e0[:, 2 * H + 1:], W_c0[:, 2 * H + 1:]],
                          axis=0).T.astype(bf16)
    w1r = jnp.concatenate([W_e0[:, 2 * H], W_c0[:, 2 * H]])[None, :]
    b1 = jnp.concatenate([b_e0, b_c0])[None, :]
    we1t = W_e1.T.astype(bf16)
    be1 = b_e1[None, :]
    watt = W_att
    batt = b_att[None, :]
    wc1t = W_c1.T.astype(bf16)
    bc1 = b_c1[None, :]
    wc2 = W_c2
    a_bf = a.astype(bf16)

    BE = 2000
    n_eb = E // BE
    full = lambda shape: pl.BlockSpec(shape, lambda i: (0,) * len(shape))
    eb = lambda w: pl.BlockSpec((BE, w), lambda i: (i, 0))
    msgh, msgx = pl.pallas_call(
        _edge_block_kernel,
        grid=(n_eb,),
        in_specs=[
            eb(H), eb(H), eb(H), eb(DE),
            full((H, 2 * H)), full((H, 2 * H)), full((DE, 2 * H)),
            full((1, 2 * H)), full((1, 2 * H)),
            full((H, H)), full((1, H)), full((1, H)), full((1, 1)),
            full((H, H)), full((1, H)), full((1, H)),
        ],
        out_specs=[eb(H), eb(H)],
        out_shape=[
            jax.ShapeDtypeStruct((E, H), f32),
            jax.ShapeDtypeStruct((E, H), f32),
        ],
    )(hs, hd, d, a_bf, w1s, w1d, w1a, w1r, b1,
      we1t, be1, watt, batt, wc1t, bc1, wc2)

    zh = jnp.zeros((N, H), f32)
    hagg, xagg = _make_scatter(N, E, H)(msgh, msgx, dst, zh)

    wn0h = W_n0[:, :H].T
    wn0a = W_n0[:, H:].T
    bn0 = b_n0[None, :]
    wn1t = W_n1.T
    bn1 = b_n1[None, :]
    BN = 2000
    n_nb = N // BN
    nb = lambda w: pl.BlockSpec((BN, w), lambda i: (i, 0))
    hout, cout128 = pl.pallas_call(
        _node_block_kernel,
        grid=(n_nb,),
        in_specs=[
            nb(H), nb(H), nb(H), nb(H),
            full((H, H)), full((H, H)), full((1, H)),
            full((H, H)), full((1, H)),
        ],
        out_specs=[nb(H), nb(H)],
        out_shape=[
            jax.ShapeDtypeStruct((N, H), f32),
            jax.ShapeDtypeStruct((N, H), f32),
        ],
    )(h, c128, hagg, xagg, wn0h, wn0a, bn0, wn1t, bn1)

    return hout, cout128[:, :coords.shape[1]]


# speedup vs baseline: 5.4255x; 1.1342x over previous
import functools

import jax
import jax.numpy as jnp
from jax import lax
from jax.experimental import pallas as pl
from jax.experimental.pallas import tpu as pltpu
from jax.experimental.pallas import tpu_sc as plsc

NC = 2
NS = 16
NW = NC * NS
CH = 80


def _make_gather(N, E, H):
    per_w = E // NW
    n_ch = per_w // CH
    mesh = plsc.VectorSubcoreMesh(core_axis_name="c", subcore_axis_name="s")

    @functools.partial(
        pl.kernel,
        out_type=(
            jax.ShapeDtypeStruct((E, H), jnp.float32),
            jax.ShapeDtypeStruct((E, H), jnp.float32),
            jax.ShapeDtypeStruct((E, H), jnp.float32),
        ),
        mesh=mesh,
        scratch_types=[
            [pltpu.VMEM((CH,), jnp.int32)] * 2,
            [pltpu.VMEM((CH,), jnp.int32)] * 2,
            [pltpu.VMEM((CH, H), jnp.float32)] * 2,
            [pltpu.VMEM((CH, H), jnp.float32)] * 2,
            [pltpu.VMEM((CH, H), jnp.float32)] * 2,
            [pltpu.VMEM((CH, H), jnp.float32)] * 2,
            [pltpu.SemaphoreType.DMA] * 2,
            [pltpu.SemaphoreType.DMA] * 2,
            [pltpu.SemaphoreType.DMA] * 2,
        ],
    )
    def gather_k(h_hbm, c128_hbm, src_hbm, dst_hbm,
                 hs_out, hd_out, d_out,
                 sidx, didx, hs_b, hd_b, cs_b, cd_b,
                 sem_l, sem_g, sem_w):
        wid = lax.axis_index("s") * NC + lax.axis_index("c")
        base0 = wid * per_w

        def fire_l(b, j):
            base = base0 + j * CH
            pltpu.async_copy(src_hbm.at[pl.ds(base, CH)], sidx[b], sem_l[b])
            pltpu.async_copy(dst_hbm.at[pl.ds(base, CH)], didx[b], sem_l[b])

        def wait_l(b):
            pltpu.make_async_copy(src_hbm.at[pl.ds(0, CH)], sidx[b],
                                  sem_l[b]).wait()
            pltpu.make_async_copy(dst_hbm.at[pl.ds(0, CH)], didx[b],
                                  sem_l[b]).wait()

        def fire_g(b):
            pltpu.async_copy(h_hbm.at[sidx[b]], hs_b[b], sem_g[b])
            pltpu.async_copy(h_hbm.at[didx[b]], hd_b[b], sem_g[b])
            pltpu.async_copy(c128_hbm.at[sidx[b]], cs_b[b], sem_g[b])
            pltpu.async_copy(c128_hbm.at[didx[b]], cd_b[b], sem_g[b])

        def wait_g(b):
            pltpu.make_async_copy(h_hbm.at[sidx[b]], hs_b[b], sem_g[b]).wait()
            pltpu.make_async_copy(h_hbm.at[didx[b]], hd_b[b], sem_g[b]).wait()
            pltpu.make_async_copy(c128_hbm.at[sidx[b]], cs_b[b],
                                  sem_g[b]).wait()
            pltpu.make_async_copy(c128_hbm.at[didx[b]], cd_b[b],
                                  sem_g[b]).wait()

        def diffs(b):
            def sub_row(i, c):
                cs_b[b][i, pl.ds(0, 16)] = (cs_b[b][i, pl.ds(0, 16)]
                                            - cd_b[b][i, pl.ds(0, 16)])
                return c
            lax.fori_loop(0, CH, sub_row, 0)

        def fire_w(b, j):
            base = base0 + j * CH
            pltpu.async_copy(hs_b[b], hs_out.at[pl.ds(base, CH)], sem_w[b])
            pltpu.async_copy(hd_b[b], hd_out.at[pl.ds(base, CH)], sem_w[b])
            pltpu.async_copy(cs_b[b], d_out.at[pl.ds(base, CH)], sem_w[b])

        def wait_w(b):
            z = pl.ds(0, CH)
            pltpu.make_async_copy(hs_b[b], hs_out.at[z], sem_w[b]).wait()
            pltpu.make_async_copy(hd_b[b], hd_out.at[z], sem_w[b]).wait()
            pltpu.make_async_copy(cs_b[b], d_out.at[z], sem_w[b]).wait()

        fire_l(0, 0)
        wait_l(0); fire_g(0)
        fire_l(1, 1)
        wait_g(0); fire_l(0, 2); diffs(0); fire_w(0, 0)
        wait_l(1); fire_g(1)
        wait_g(1); fire_l(1, 3); diffs(1); fire_w(1, 1)

        n_pairs = (n_ch - 2) // 2
        last = n_ch - 1

        def body(k, carry):
            for b in (0, 1):
                j = 2 * k + b
                wait_l(b)
                wait_w(b)
                fire_g(b)
                wait_g(b)
                jn = jnp.minimum(j + 2, last)
                fire_l(b, jn)
                diffs(b)
                fire_w(b, j)
            return carry

        lax.fori_loop(1, 1 + n_pairs, body, 0)

        if n_ch % 2:
            wait_l(0)
            wait_w(0)
            fire_g(0)
            wait_g(0)
            diffs(0)
            fire_w(0, last)
            wait_l(1)
            wait_w(1)
            wait_w(0)
        else:
            wait_l(0); wait_l(1)
            wait_w(0); wait_w(1)

    return gather_k


def _make_scatter(N, E, H):
    per_t = E // NS
    n_ch = per_t // CH
    rpt = (N // NS) // 8 * 8
    rem = N - NS * rpt
    mesh = plsc.VectorSubcoreMesh(core_axis_name="c", subcore_axis_name="s")

    @functools.partial(
        pl.kernel,
        out_type=(
            jax.ShapeDtypeStruct((N, H), jnp.float32),
            jax.ShapeDtypeStruct((N, H), jnp.float32),
        ),
        mesh=mesh,
        scratch_types=[
            [pltpu.VMEM((CH,), jnp.int32)] * 2,
            [pltpu.VMEM((CH, H), jnp.float32)] * 2,
            pltpu.VMEM_SHARED((N, H), jnp.float32),
            [pltpu.SemaphoreType.DMA] * 2,
            [pltpu.SemaphoreType.DMA] * 2,
        ],
    )
    def scatter_k(msgh_hbm, msgx_hbm, dst_hbm, zh_hbm,
                  hagg_out, xagg_out,
                  didx, m_b, acc, sem_l, sem_a):
        cid = lax.axis_index("c")
        sid = lax.axis_index("s")
        base0 = sid * per_t
        r0 = sid * rpt
        pltpu.sync_copy(zh_hbm.at[pl.ds(r0, rpt)], acc.at[pl.ds(r0, rpt)])
        if rem:
            @pl.when(sid == NS - 1)
            def _():
                pltpu.sync_copy(zh_hbm.at[pl.ds(NS * rpt, rem)],
                                acc.at[pl.ds(NS * rpt, rem)])
        plsc.subcore_barrier()

        def run_pipeline(src_ref):
            def fire_l(b, j):
                base = base0 + j * CH
                pltpu.async_copy(dst_hbm.at[pl.ds(base, CH)], didx[b],
                                 sem_l[b])
                pltpu.async_copy(src_ref.at[pl.ds(base, CH)], m_b[b],
                                 sem_l[b])

            def wait_l(b):
                pltpu.make_async_copy(dst_hbm.at[pl.ds(0, CH)], didx[b],
                                      sem_l[b]).wait()
                pltpu.make_async_copy(src_ref.at[pl.ds(0, CH)], m_b[b],
                                      sem_l[b]).wait()

            def fire_a(b):
                pltpu.async_copy(m_b[b], acc.at[didx[b]], sem_a[b], add=True)

            def wait_a(b):
                pltpu.make_async_copy(m_b[b], acc.at[didx[b]],
                                      sem_a[b]).wait()

            fire_l(0, 0)
            fire_l(1, 1)
            last = n_ch - 1

            def body(k, carry):
                for b in (0, 1):
                    j = 2 * k + b
                    wait_l(b)
                    fire_a(b)
                    wait_a(b)
                    jn = jnp.minimum(j + 2, last)
                    fire_l(b, jn)
                return carry

            lax.fori_loop(0, n_ch // 2, body, 0)
            wait_l(0)
            wait_l(1)

        @pl.when(cid == 0)
        def _():
            run_pipeline(msgh_hbm)

        @pl.when(cid == 1)
        def _():
            run_pipeline(msgx_hbm)

        plsc.subcore_barrier()

        @pl.when(cid == 0)
        def _():
            pltpu.sync_copy(acc.at[pl.ds(r0, rpt)],
                            hagg_out.at[pl.ds(r0, rpt)])
            if rem:
                @pl.when(sid == NS - 1)
                def _():
                    pltpu.sync_copy(acc.at[pl.ds(NS * rpt, rem)],
                                    hagg_out.at[pl.ds(NS * rpt, rem)])

        @pl.when(cid == 1)
        def _():
            pltpu.sync_copy(acc.at[pl.ds(r0, rpt)],
                            xagg_out.at[pl.ds(r0, rpt)])
            if rem:
                @pl.when(sid == NS - 1)
                def _():
                    pltpu.sync_copy(acc.at[pl.ds(NS * rpt, rem)],
                                    xagg_out.at[pl.ds(NS * rpt, rem)])

    return scatter_k


def _silu2(xh):
    return xh * (jnp.tanh(xh) + 1.0)


def _edge_block_kernel(hs, hd, d_ref, a_ref,
                       w1s, w1d, w1a, w1r, b1,
                       we1t, be1, watt, batt,
                       wc1t, bc1, wc2,
                       msgh_out, msgx_out):
    H = hs.shape[1]
    bf16 = jnp.bfloat16
    f32 = jnp.float32
    hs_ = hs[...].astype(bf16)
    hd_ = hd[...].astype(bf16)
    d = d_ref[...]
    r2 = jnp.sum(d * d, axis=1, keepdims=True)
    r = jnp.sqrt(r2)
    pre = (jnp.dot(hs_, w1s[...], preferred_element_type=f32)
           + jnp.dot(hd_, w1d[...], preferred_element_type=f32)
           + jnp.dot(a_ref[...], w1a[...], preferred_element_type=f32)
           + r * w1r[...] + b1[...])
    m_e = _silu2(pre[:, :H]).astype(bf16)
    mh = _silu2(jnp.dot(m_e, we1t[...],
                        preferred_element_type=f32) + be1[...])
    att_t = jnp.tanh(
        jnp.sum(mh * watt[...], axis=1, keepdims=True) + batt[0, 0]) + 1.0
    msgh_out[...] = att_t * mh
    m1 = _silu2(pre[:, H:]).astype(bf16)
    m2 = _silu2(jnp.dot(m1, wc1t[...],
                        preferred_element_type=f32) + bc1[...])
    s = jnp.sum(m2 * wc2[...], axis=1, keepdims=True)
    msgx_out[...] = s * d / (r + 1.0)


def _node_block_kernel(h, c128, hagg, xagg,
                       wn0h, wn0a, bn0, wn1t, bn1,
                       hout, cout):
    h_ = h[...]
    t = _silu2(jnp.dot(h_, wn0h[...], preferred_element_type=jnp.float32)
               + jnp.dot(hagg[...], wn0a[...],
                         preferred_element_type=jnp.float32)
               + bn0[...])
    hout[...] = h_ + jnp.dot(t, wn1t[...],
                             preferred_element_type=jnp.float32) + bn1[...]
    cout[...] = c128[...] + xagg[...]


def kernel(h, coords, a, edge_index, W_e0, b_e0, W_e1, b_e1, W_att, b_att,
           W_n0, b_n0, W_n1, b_n1, W_c0, b_c0, W_c1, b_c1, W_c2):
    N, H = h.shape
    E = a.shape[0]
    DE = a.shape[1]
    f32 = jnp.float32

    bf16 = jnp.bfloat16
    src = edge_index[0]
    dst = edge_index[1]
    c128 = jnp.pad(coords, ((0, 0), (0, H - coords.shape[1])))

    hs, hd, d = _make_gather(N, E, H)(h, c128, src, dst)

    w1s = (0.5 * jnp.concatenate([W_e0[:, :H], W_c0[:, :H]],
                                 axis=0).T).astype(bf16)
    w1d = (0.5 * jnp.concatenate([W_e0[:, H:2 * H], W_c0[:, H:2 * H]],
                                 axis=0).T).astype(bf16)
    w1a = (0.5 * jnp.concatenate([W_e0[:, 2 * H + 1:], W_c0[:, 2 * H + 1:]],
                                 axis=0).T).astype(bf16)
    w1r = 0.5 * jnp.concatenate([W_e0[:, 2 * H],
                                 W_c0[:, 2 * H]])[None, :]
    b1 = 0.5 * jnp.concatenate([b_e0, b_c0])[None, :]
    we1t = (0.5 * W_e1.T).astype(bf16)
    be1 = 0.5 * b_e1[None, :]
    watt = 0.5 * W_att
    batt = 0.5 * b_att[None, :]
    wc1t = (0.5 * W_c1.T).astype(bf16)
    bc1 = 0.5 * b_c1[None, :]
    wc2 = W_c2
    a_bf = a.astype(bf16)

    BE = 4000
    n_eb = E // BE
    full = lambda shape: pl.BlockSpec(shape, lambda i: (0,) * len(shape))
    eb = lambda w: pl.BlockSpec((BE, w), lambda i: (i, 0))
    msgh, msgx = pl.pallas_call(
        _edge_block_kernel,
        grid=(n_eb,),
        in_specs=[
            eb(H), eb(H), eb(H), eb(DE),
            full((H, 2 * H)), full((H, 2 * H)), full((DE, 2 * H)),
            full((1, 2 * H)), full((1, 2 * H)),
            full((H, H)), full((1, H)), full((1, H)), full((1, 1)),
            full((H, H)), full((1, H)), full((1, H)),
        ],
        out_specs=[eb(H), eb(H)],
        out_shape=[
            jax.ShapeDtypeStruct((E, H), f32),
            jax.ShapeDtypeStruct((E, H), f32),
        ],
    )(hs, hd, d, a_bf, w1s, w1d, w1a, w1r, b1,
      we1t, be1, watt, batt, wc1t, bc1, wc2)

    zh = jnp.zeros((N, H), f32)
    hagg, xagg = _make_scatter(N, E, H)(msgh, msgx, dst, zh)

    wn0h = 0.5 * W_n0[:, :H].T
    wn0a = 0.25 * W_n0[:, H:].T
    bn0 = 0.5 * b_n0[None, :]
    wn1t = W_n1.T
    bn1 = b_n1[None, :]
    BN = 2000
    n_nb = N // BN
    nb = lambda w: pl.BlockSpec((BN, w), lambda i: (i, 0))
    hout, cout128 = pl.pallas_call(
        _node_block_kernel,
        grid=(n_nb,),
        in_specs=[
            nb(H), nb(H), nb(H), nb(H),
            full((H, H)), full((H, H)), full((1, H)),
            full((H, H)), full((1, H)),
        ],
        out_specs=[nb(H), nb(H)],
        out_shape=[
            jax.ShapeDtypeStruct((N, H), f32),
            jax.ShapeDtypeStruct((N, H), f32),
        ],
    )(h, c128, hagg, xagg, wn0h, wn0a, bn0, wn1t, bn1)

    return hout, cout128[:, :coords.shape[1]]
